# baseline (speedup 1.0000x reference)
import functools

import jax
import jax.numpy as jnp
from jax import lax
from jax.experimental import pallas as pl
from jax.experimental.pallas import tpu as pltpu
from jax.experimental.pallas import tpu_sc as plsc

B = 1024
F = 1000
BIN = 10
D = 16
MAX_NORM = 1.0

NC, NS, L = 2, 16, 16
NW = NC * NS
NWIN = 8
NSPLIT = 4
TW = 128
FR = 200
NU = BIN * (F // FR)
VP = 1024
CB = (TW // 2) * VP


def _histogram_sc(idx_t):
    mesh = plsc.VectorSubcoreMesh(core_axis_name="c", subcore_axis_name="s")

    @functools.partial(
        pl.kernel,
        mesh=mesh,
        out_type=jax.ShapeDtypeStruct((NW * CB,), jnp.int32),
        scratch_types=[
            pltpu.VMEM((2, FR, TW), jnp.int32),
            pltpu.VMEM((CB,), jnp.int32),
            pltpu.SemaphoreType.DMA((2,)),
        ],
        compiler_params=pltpu.CompilerParams(needs_layout_passes=False),
    )
    def hist_kernel(idx_hbm, counts_hbm, stg, counts_v, sems):
        wid = lax.axis_index("s") * NC + lax.axis_index("c")
        wi = wid % NWIN
        s = wid // NWIN
        bwin = wi * TW
        nch = jnp.where(s < 2, 13, 12)
        ubase = jnp.where(s < 2, 13 * s, 12 * s + 2)
        zeros = jnp.zeros((L,), jnp.int32)
        lanes = lax.iota(jnp.int32, L)
        alt = (lanes & 1) * 65535 + 1

        def start(t):
            u = ubase + t
            j = u // (F // FR)
            f0 = (u % (F // FR)) * FR
            buf = t % 2
            pltpu.make_async_copy(
                idx_hbm.at[j, pl.ds(f0, FR), pl.ds(bwin, TW)],
                stg.at[buf],
                sems.at[buf],
            ).start()

        start(jnp.int32(0))

        @plsc.parallel_loop(0, CB // L, unroll=8)
        def _zero(k):
            counts_v[pl.ds(k * L, L)] = zeros

        bases = [((jnp.full((L,), g * L, jnp.int32) + lanes) // 2) * VP
                 for g in range(TW // L)]

        def chunk_body(t, _):
            @pl.when(t + 1 < nch)
            def _start_next():
                start(t + 1)

            buf = t % 2
            pltpu.make_async_copy(
                idx_hbm.at[0, pl.ds(0, FR), pl.ds(0, TW)],
                stg.at[buf],
                sems.at[buf],
            ).wait()

            @plsc.parallel_loop(0, FR, unroll=8)
            def _rows(r):
                for g in range(TW // L):
                    iv = stg[buf, r, pl.ds(g * L, L)]
                    plsc.addupdate_scatter(counts_v, [bases[g] + iv], alt)

            return _

        lax.fori_loop(0, nch, chunk_body, None)

        w2 = wi * NSPLIT + s
        pltpu.sync_copy(counts_v, counts_hbm.at[pl.ds(w2 * CB, CB)])

    return hist_kernel(idx_t)


def _finish_tc(partials, pos_table, val_pad):
    def body(p_ref, pos_ref, val_ref, out_ref):
        def renorm(t):
            n = jnp.sqrt(jnp.sum(t * t, axis=1, keepdims=True))
            return t * jnp.minimum(1.0, MAX_NORM / jnp.maximum(n, 1e-12))

        val_r = renorm(val_ref[...])
        pos_r = renorm(pos_ref[...])
        pos_mean = jnp.sum(pos_r, axis=0, keepdims=True) * (1.0 / F)
        p = p_ref[...].reshape(NSPLIT, TW // 2, VP // TW, TW)
        ps = p.sum(axis=0)
        nrows = TW // 2
        out_even = jnp.zeros((nrows, D), jnp.float32)
        out_odd = jnp.zeros((nrows, D), jnp.float32)
        for vb in range(VP // TW):
            blk = ps[:, vb, :]
            low = (blk & 0xFFFF).astype(jnp.float32)
            high = lax.shift_right_logical(blk, 16).astype(jnp.float32)
            vrows = val_r[vb * TW:(vb + 1) * TW, :]
            out_even = out_even + jnp.dot(low, vrows, preferred_element_type=jnp.float32)
            out_odd = out_odd + jnp.dot(high, vrows, preferred_element_type=jnp.float32)
        both = jnp.stack([out_even, out_odd], axis=1)
        out = both.reshape(TW, D) * (1.0 / F) + pos_mean
        out_ref[...] = out.T

    return pl.pallas_call(
        body,
        grid=(NWIN,),
        in_specs=[
            pl.BlockSpec((NSPLIT * (TW // 2) * (VP // TW), TW), lambda w: (w, 0)),
            pl.BlockSpec((F, D), lambda w: (0, 0)),
            pl.BlockSpec((VP, D), lambda w: (0, 0)),
        ],
        out_specs=pl.BlockSpec((D, TW), lambda w: (0, w)),
        out_shape=jax.ShapeDtypeStruct((D, B), jnp.float32),
    )(partials, pos_table, val_pad)


def kernel(input_tensor, pos_table, val_table):
    idx_t = input_tensor.transpose(2, 1, 0)
    partials = _histogram_sc(idx_t).reshape(NW * CB // TW, TW)
    val_pad = jnp.pad(val_table, ((0, VP - F), (0, 0)))
    return _finish_tc(partials, pos_table, val_pad).T


# speedup vs baseline: 1.0379x; 1.0379x over previous
import functools

import jax
import jax.numpy as jnp
from jax import lax
from jax.experimental import pallas as pl
from jax.experimental.pallas import tpu as pltpu
from jax.experimental.pallas import tpu_sc as plsc

B = 1024
F = 1000
BIN = 10
D = 16
MAX_NORM = 1.0

NC, NS, L = 2, 16, 16
NW = NC * NS
NWIN = 8
NSPLIT = 4
TW = 128
FR = 200
NU = BIN * (F // FR)
VP = 1024
CB = (TW // 2) * VP


def _histogram_sc(idx_t):
    mesh = plsc.VectorSubcoreMesh(core_axis_name="c", subcore_axis_name="s")

    @functools.partial(
        pl.kernel,
        mesh=mesh,
        out_type=jax.ShapeDtypeStruct((NW * CB,), jnp.int32),
        scratch_types=[
            pltpu.VMEM((2, FR, TW), jnp.int32),
            pltpu.VMEM((CB,), jnp.int32),
            pltpu.SemaphoreType.DMA((2,)),
        ],
        compiler_params=pltpu.CompilerParams(needs_layout_passes=False),
    )
    def hist_kernel(idx_hbm, counts_hbm, stg, counts_v, sems):
        wid = lax.axis_index("s") * NC + lax.axis_index("c")
        wi = wid % NWIN
        s = wid // NWIN
        bwin = wi * TW
        nch = jnp.where(s < 2, 13, 12)
        ubase = jnp.where(s < 2, 13 * s, 12 * s + 2)
        zeros = jnp.zeros((L,), jnp.int32)
        lanes = lax.iota(jnp.int32, L)
        alt = (lanes & 1) * 65535 + 1

        def start(t):
            u = ubase + t
            j = u // (F // FR)
            f0 = (u % (F // FR)) * FR
            buf = t % 2
            pltpu.make_async_copy(
                idx_hbm.at[j, pl.ds(f0, FR), pl.ds(bwin, TW)],
                stg.at[buf],
                sems.at[buf],
            ).start()

        start(jnp.int32(0))

        @plsc.parallel_loop(0, CB // L, unroll=8)
        def _zero(k):
            counts_v[pl.ds(k * L, L)] = zeros

        bases = [((jnp.full((L,), g * L, jnp.int32) + lanes) // 2) * VP
                 for g in range(TW // L)]

        def chunk_body(t, _):
            @pl.when(t + 1 < nch)
            def _start_next():
                start(t + 1)

            buf = t % 2
            pltpu.make_async_copy(
                idx_hbm.at[0, pl.ds(0, FR), pl.ds(0, TW)],
                stg.at[buf],
                sems.at[buf],
            ).wait()

            @plsc.parallel_loop(0, FR, unroll=8)
            def _rows(r):
                for g in range(TW // L):
                    iv = stg[buf, r, pl.ds(g * L, L)]
                    plsc.addupdate_scatter(counts_v, [bases[g] + iv], alt)

            return _

        lax.fori_loop(0, nch, chunk_body, None)

        w2 = wi * NSPLIT + s
        pltpu.sync_copy(counts_v, counts_hbm.at[pl.ds(w2 * CB, CB)])

    return hist_kernel(idx_t)


def _finish_tc(partials, pos_table, val_pad):
    def body(p_ref, pos_ref, val_ref, out_ref):
        def renorm(t):
            n = jnp.sqrt(jnp.sum(t * t, axis=1, keepdims=True))
            return t * jnp.minimum(1.0, MAX_NORM / jnp.maximum(n, 1e-12))

        val_r = renorm(val_ref[...])
        pos_r = renorm(pos_ref[...])
        pos_mean = jnp.sum(pos_r, axis=0, keepdims=True) * (1.0 / F)
        p = p_ref[...].reshape(NWIN, NSPLIT, TW // 2, VP // TW, TW)
        ps = p.sum(axis=1)
        nrows = NWIN * (TW // 2)
        out_even = jnp.zeros((nrows, D), jnp.float32)
        out_odd = jnp.zeros((nrows, D), jnp.float32)
        for vb in range(VP // TW):
            blk = ps[:, :, vb, :].reshape(nrows, TW)
            low = (blk & 0xFFFF).astype(jnp.float32)
            high = lax.shift_right_logical(blk, 16).astype(jnp.float32)
            vrows = val_r[vb * TW:(vb + 1) * TW, :]
            out_even = out_even + jnp.dot(low, vrows, preferred_element_type=jnp.float32)
            out_odd = out_odd + jnp.dot(high, vrows, preferred_element_type=jnp.float32)
        both = jnp.stack([out_even, out_odd], axis=1)
        out = both.reshape(B, D) * (1.0 / F) + pos_mean
        out_ref[...] = out.T

    return pl.pallas_call(
        body,
        out_shape=jax.ShapeDtypeStruct((D, B), jnp.float32),
    )(partials, pos_table, val_pad)


def kernel(input_tensor, pos_table, val_table):
    idx_t = input_tensor.transpose(2, 1, 0)
    partials = _histogram_sc(idx_t).reshape(NW * CB // TW, TW)
    val_pad = jnp.pad(val_table, ((0, VP - F), (0, 0)))
    return _finish_tc(partials, pos_table, val_pad).T
